# Initial kernel scaffold; baseline (speedup 1.0000x reference)
import jax
import jax.numpy as jnp
from jax.experimental import pallas as pl


def kernel(question, image, wembed, gru_w_ih, gru_w_hh, gru_b_ih, gru_b_hh, W1, b1, W2, b2, W3, b3, W4, b4, W5, b5):
    raise NotImplementedError("write your pallas kernel here")


# speedup vs baseline: 2.7007x; 2.7007x over previous
import functools

import jax
import jax.numpy as jnp
from jax import lax
from jax.experimental import pallas as pl
from jax.experimental.pallas import tpu as pltpu, tpu_sc as plsc

VOCAB = 20000
EMB = 1024
HID = 1024
FEAT = 1024
K = 36
OUT = 3000
B = 512
S = 20

F32 = jnp.float32


def _sc_gather(table, idx):
    n = idx.shape[0]
    info = plsc.get_sparse_core_info()
    nc, ns = info.num_cores, info.num_subcores
    nw = nc * ns
    assert n % nw == 0
    b_per_w = n // nw
    chunk = 64
    assert b_per_w % chunk == 0
    n_chunks = b_per_w // chunk

    mesh = plsc.VectorSubcoreMesh(core_axis_name="c", subcore_axis_name="s")

    @functools.partial(
        pl.kernel,
        mesh=mesh,
        out_type=jax.ShapeDtypeStruct((n, EMB), F32),
        scratch_types=[
            pltpu.VMEM((chunk,), jnp.int32),
            pltpu.VMEM((chunk, EMB), F32),
            pltpu.SemaphoreType.DMA,
        ],
    )
    def gather_k(idx_hbm, table_hbm, out_hbm, idx_v, rows_v, sem):
        wid = lax.axis_index("s") * nc + lax.axis_index("c")
        base = wid * b_per_w
        for c in range(n_chunks):
            off = base + c * chunk
            pltpu.sync_copy(idx_hbm.at[pl.ds(off, chunk)], idx_v)
            pltpu.async_copy(table_hbm.at[idx_v], rows_v, sem).wait()
            pltpu.sync_copy(rows_v, out_hbm.at[pl.ds(off, chunk)])

    return gather_k(idx, table)


def _gru_body(x_ref, wih_ref, whh_ref, bih_ref, bhh_ref, out_ref, h_ref):
    t = pl.program_id(0)

    @pl.when(t == 0)
    def _():
        h_ref[...] = jnp.zeros_like(h_ref)

    h = h_ref[...]
    x_t = x_ref[0]
    gi = jnp.dot(x_t, wih_ref[...], preferred_element_type=F32) + bih_ref[0]
    gh = jnp.dot(h, whh_ref[...], preferred_element_type=F32) + bhh_ref[0]
    r = jax.nn.sigmoid(gi[:, :HID] + gh[:, :HID])
    z = jax.nn.sigmoid(gi[:, HID:2 * HID] + gh[:, HID:2 * HID])
    n = jnp.tanh(gi[:, 2 * HID:] + r * gh[:, 2 * HID:])
    hn = (1.0 - z) * n + z * h
    h_ref[...] = hn
    out_ref[...] = hn


def _gru(x, wih_t, whh_t, b_ih, b_hh):
    return pl.pallas_call(
        _gru_body,
        grid=(S,),
        in_specs=[
            pl.BlockSpec((1, B, EMB), lambda t: (t, 0, 0)),
            pl.BlockSpec((EMB, 3 * HID), lambda t: (0, 0)),
            pl.BlockSpec((HID, 3 * HID), lambda t: (0, 0)),
            pl.BlockSpec((1, 3 * HID), lambda t: (0, 0)),
            pl.BlockSpec((1, 3 * HID), lambda t: (0, 0)),
        ],
        out_specs=pl.BlockSpec((B, HID), lambda t: (0, 0)),
        out_shape=jax.ShapeDtypeStruct((B, HID), F32),
        scratch_shapes=[pltpu.VMEM((B, HID), F32)],
        compiler_params=pltpu.CompilerParams(
            dimension_semantics=("arbitrary",),
        ),
    )(x, wih_t, whh_t, b_ih, b_hh)


_BB = 32


def _attn_body(img_ref, q_ref, w1a_ref, w1b_ref, b1_ref, w2_ref, ctx_ref):
    x = img_ref[...]
    recip = 1.0 / jnp.abs(x)
    nsum = jnp.sum(recip, axis=0)
    norm = jnp.maximum(1.0 / nsum, 1e-12)
    img = x / norm[None]

    q = q_ref[...]
    qw = jnp.dot(q, w1b_ref[...], preferred_element_type=F32)
    img2 = img.reshape(K * _BB, FEAT)
    c1 = jnp.dot(img2, w1a_ref[...], preferred_element_type=F32)
    c1 = c1.reshape(K, _BB, HID) + qw[None] + b1_ref[0][None, None]
    c1 = jax.nn.relu(c1)

    att = jnp.sum(c1 * w2_ref[0][None, None, :], axis=2)
    att = att - jnp.max(att, axis=0)[None]
    e = jnp.exp(att)
    att = e / jnp.sum(e, axis=0)[None]

    ctx_ref[...] = jnp.sum(att[:, :, None] * img, axis=0)


def _attn(img_t, q_enc, w1a, w1b, b1, w2row):
    nblk = B // _BB
    const = lambda i: (0, 0)
    return pl.pallas_call(
        _attn_body,
        grid=(nblk,),
        in_specs=[
            pl.BlockSpec((K, _BB, FEAT), lambda i: (0, i, 0)),
            pl.BlockSpec((_BB, HID), lambda i: (i, 0)),
            pl.BlockSpec((FEAT, HID), const),
            pl.BlockSpec((HID, HID), const),
            pl.BlockSpec((1, HID), const),
            pl.BlockSpec((1, HID), const),
        ],
        out_specs=pl.BlockSpec((_BB, FEAT), lambda i: (i, 0)),
        out_shape=jax.ShapeDtypeStruct((B, FEAT), F32),
        compiler_params=pltpu.CompilerParams(
            dimension_semantics=("parallel",),
        ),
    )(img_t, q_enc, w1a, w1b, b1, w2row)


def _head_body(ctx_ref, q_ref, w3a_ref, w3b_ref, b3_ref, w4_ref, b4_ref,
               w5_ref, b5_ref, out_ref):
    h1 = jax.nn.relu(
        jnp.dot(ctx_ref[...], w3a_ref[...], preferred_element_type=F32)
        + jnp.dot(q_ref[...], w3b_ref[...], preferred_element_type=F32)
        + b3_ref[0]
    )
    h2 = jax.nn.relu(jnp.dot(h1, w4_ref[...], preferred_element_type=F32)
                     + b4_ref[0])
    out_ref[...] = (jnp.dot(h2, w5_ref[...], preferred_element_type=F32)
                    + b5_ref[0])


def _head(ctx, q_enc, w3a, w3b, b3, w4, b4, w5, b5):
    return pl.pallas_call(
        _head_body,
        out_shape=jax.ShapeDtypeStruct((B, OUT), F32),
    )(ctx, q_enc, w3a, w3b, b3, w4, b4, w5, b5)


def kernel(question, image, wembed, gru_w_ih, gru_w_hh, gru_b_ih, gru_b_hh,
           W1, b1, W2, b2, W3, b3, W4, b4, W5, b5):
    idx = question.T.reshape(-1).astype(jnp.int32)
    img_t = jnp.transpose(image, (1, 0, 2))
    wih_t = gru_w_ih.T
    whh_t = gru_w_hh.T
    bih = gru_b_ih.reshape(1, 3 * HID)
    bhh = gru_b_hh.reshape(1, 3 * HID)
    w1a = W1[:FEAT]
    w1b = W1[FEAT:]
    w2row = W2.reshape(1, HID)
    w3a = W3[:FEAT]
    w3b = W3[FEAT:]
    b1r = b1.reshape(1, HID)
    b3r = b3.reshape(1, HID)
    b4r = b4.reshape(1, HID)
    b5r = b5.reshape(1, OUT)

    emb = _sc_gather(wembed, idx).reshape(S, B, EMB)

    q_enc = _gru(emb, wih_t, whh_t, bih, bhh)

    ctx = _attn(img_t, q_enc, w1a, w1b, b1r, w2row)
    return _head(ctx, q_enc, w3a, w3b, b3r, W4, b4r, W5, b5r)


# speedup vs baseline: 2.7388x; 1.0141x over previous
import functools

import jax
import jax.numpy as jnp
from jax import lax
from jax.experimental import pallas as pl
from jax.experimental.pallas import tpu as pltpu, tpu_sc as plsc

VOCAB = 20000
EMB = 1024
HID = 1024
FEAT = 1024
K = 36
OUT = 3000
B = 512
S = 20

F32 = jnp.float32
BF16 = jnp.bfloat16


def _sc_gather(table, idx):
    n = idx.shape[0]
    info = plsc.get_sparse_core_info()
    nc, ns = info.num_cores, info.num_subcores
    nw = nc * ns
    assert n % nw == 0
    b_per_w = n // nw
    chunk = 64
    assert b_per_w % chunk == 0
    n_chunks = b_per_w // chunk

    mesh = plsc.VectorSubcoreMesh(core_axis_name="c", subcore_axis_name="s")

    @functools.partial(
        pl.kernel,
        mesh=mesh,
        out_type=jax.ShapeDtypeStruct((n, EMB), F32),
        scratch_types=[
            pltpu.VMEM((chunk,), jnp.int32),
            pltpu.VMEM((chunk, EMB), F32),
            pltpu.SemaphoreType.DMA,
        ],
    )
    def gather_k(idx_hbm, table_hbm, out_hbm, idx_v, rows_v, sem):
        wid = lax.axis_index("s") * nc + lax.axis_index("c")
        base = wid * b_per_w
        for c in range(n_chunks):
            off = base + c * chunk
            pltpu.sync_copy(idx_hbm.at[pl.ds(off, chunk)], idx_v)
            pltpu.async_copy(table_hbm.at[idx_v], rows_v, sem).wait()
            pltpu.sync_copy(rows_v, out_hbm.at[pl.ds(off, chunk)])

    return gather_k(idx, table)


def _gru_body(x_ref, wih_ref, whh_ref, bih_ref, bhh_ref, out_ref, h_ref):
    t = pl.program_id(0)

    @pl.when(t == 0)
    def _():
        h_ref[...] = jnp.zeros_like(h_ref)

    h = h_ref[...]
    x_t = x_ref[0].astype(BF16)
    gi = jnp.dot(x_t, wih_ref[...], preferred_element_type=F32) + bih_ref[0]
    gh = jnp.dot(h.astype(BF16), whh_ref[...],
                 preferred_element_type=F32) + bhh_ref[0]
    r = jax.nn.sigmoid(gi[:, :HID] + gh[:, :HID])
    z = jax.nn.sigmoid(gi[:, HID:2 * HID] + gh[:, HID:2 * HID])
    n = jnp.tanh(gi[:, 2 * HID:] + r * gh[:, 2 * HID:])
    hn = (1.0 - z) * n + z * h
    h_ref[...] = hn
    out_ref[...] = hn


def _gru(x, wih_t, whh_t, b_ih, b_hh):
    return pl.pallas_call(
        _gru_body,
        grid=(S,),
        in_specs=[
            pl.BlockSpec((1, B, EMB), lambda t: (t, 0, 0)),
            pl.BlockSpec((EMB, 3 * HID), lambda t: (0, 0)),
            pl.BlockSpec((HID, 3 * HID), lambda t: (0, 0)),

            pl.BlockSpec((1, 3 * HID), lambda t: (0, 0)),
            pl.BlockSpec((1, 3 * HID), lambda t: (0, 0)),
        ],
        out_specs=pl.BlockSpec((B, HID), lambda t: (0, 0)),
        out_shape=jax.ShapeDtypeStruct((B, HID), F32),
        scratch_shapes=[pltpu.VMEM((B, HID), F32)],
        compiler_params=pltpu.CompilerParams(
            dimension_semantics=("arbitrary",),
        ),
    )(x, wih_t, whh_t, b_ih, b_hh)


_BB = 32


def _attn_body(img_ref, q_ref, w1a_ref, w1b_ref, b1_ref, w2_ref, ctx_ref):
    x = img_ref[...]
    recip = 1.0 / jnp.abs(x)
    nsum = jnp.sum(recip, axis=0)
    norm = jnp.maximum(1.0 / nsum, 1e-12)
    img = x / norm[None]

    q = q_ref[...]
    qw = jnp.dot(q.astype(BF16), w1b_ref[...],
                 preferred_element_type=F32)
    img2 = img.reshape(K * _BB, FEAT).astype(BF16)
    c1 = jnp.dot(img2, w1a_ref[...], preferred_element_type=F32)
    c1 = c1.reshape(K, _BB, HID) + qw[None] + b1_ref[0][None, None]
    c1 = jax.nn.relu(c1)

    att = jnp.sum(c1 * w2_ref[0][None, None, :], axis=2)
    att = att - jnp.max(att, axis=0)[None]
    e = jnp.exp(att)
    att = e / jnp.sum(e, axis=0)[None]

    ctx_ref[...] = jnp.sum(att[:, :, None] * img, axis=0)


def _attn(img_t, q_enc, w1a, w1b, b1, w2row):
    nblk = B // _BB
    const = lambda i: (0, 0)
    return pl.pallas_call(
        _attn_body,
        grid=(nblk,),
        in_specs=[
            pl.BlockSpec((K, _BB, FEAT), lambda i: (0, i, 0)),
            pl.BlockSpec((_BB, HID), lambda i: (i, 0)),
            pl.BlockSpec((FEAT, HID), const),
            pl.BlockSpec((HID, HID), const),
            pl.BlockSpec((1, HID), const),
            pl.BlockSpec((1, HID), const),
        ],
        out_specs=pl.BlockSpec((_BB, FEAT), lambda i: (i, 0)),
        out_shape=jax.ShapeDtypeStruct((B, FEAT), F32),
        compiler_params=pltpu.CompilerParams(
            dimension_semantics=("parallel",),
        ),
    )(img_t, q_enc, w1a, w1b, b1, w2row)


def _head_body(ctx_ref, q_ref, w3a_ref, w3b_ref, b3_ref, w4_ref, b4_ref,
               w5_ref, b5_ref, out_ref):
    h1 = jax.nn.relu(
        jnp.dot(ctx_ref[...].astype(BF16), w3a_ref[...],
                preferred_element_type=F32)
        + jnp.dot(q_ref[...].astype(BF16), w3b_ref[...],
                  preferred_element_type=F32)
        + b3_ref[0]
    )
    h2 = jax.nn.relu(jnp.dot(h1.astype(BF16), w4_ref[...],
                             preferred_element_type=F32) + b4_ref[0])
    out_ref[...] = (jnp.dot(h2.astype(BF16), w5_ref[...],
                            preferred_element_type=F32) + b5_ref[0])


def _head(ctx, q_enc, w3a, w3b, b3, w4, b4, w5, b5):
    return pl.pallas_call(
        _head_body,
        out_shape=jax.ShapeDtypeStruct((B, OUT), F32),
    )(ctx, q_enc, w3a, w3b, b3, w4, b4, w5, b5)


def kernel(question, image, wembed, gru_w_ih, gru_w_hh, gru_b_ih, gru_b_hh,
           W1, b1, W2, b2, W3, b3, W4, b4, W5, b5):
    idx = question.T.reshape(-1).astype(jnp.int32)
    img_t = jnp.transpose(image, (1, 0, 2))
    wih_t = gru_w_ih.T.astype(BF16)
    whh_t = gru_w_hh.T.astype(BF16)
    bih = gru_b_ih.reshape(1, 3 * HID)
    bhh = gru_b_hh.reshape(1, 3 * HID)
    w1a = W1[:FEAT].astype(BF16)
    w1b = W1[FEAT:].astype(BF16)
    w2row = W2.reshape(1, HID)
    w3a = W3[:FEAT].astype(BF16)
    w3b = W3[FEAT:].astype(BF16)
    b1r = b1.reshape(1, HID)
    b3r = b3.reshape(1, HID)
    b4r = b4.reshape(1, HID)
    b5r = b5.reshape(1, OUT)

    emb = _sc_gather(wembed, idx).reshape(S, B, EMB)

    q_enc = _gru(emb, wih_t, whh_t, bih, bhh)

    ctx = _attn(img_t, q_enc, w1a, w1b, b1r, w2row)
    return _head(ctx, q_enc, w3a, w3b, b3r, W4.astype(BF16), b4r,
                 W5.astype(BF16), b5r)


# speedup vs baseline: 2.7891x; 1.0184x over previous
import functools

import jax
import jax.numpy as jnp
from jax import lax
from jax.experimental import pallas as pl
from jax.experimental.pallas import tpu as pltpu, tpu_sc as plsc

VOCAB = 20000
EMB = 1024
HID = 1024
FEAT = 1024
K = 36
OUT = 3000
B = 512
S = 20

F32 = jnp.float32
BF16 = jnp.bfloat16


def _sc_gather(table, idx):
    n = idx.shape[0]
    info = plsc.get_sparse_core_info()
    nc, ns = info.num_cores, info.num_subcores
    nw = nc * ns
    assert n % nw == 0
    b_per_w = n // nw
    chunk = 64
    assert b_per_w % chunk == 0
    n_chunks = b_per_w // chunk

    mesh = plsc.VectorSubcoreMesh(core_axis_name="c", subcore_axis_name="s")

    @functools.partial(
        pl.kernel,
        mesh=mesh,
        out_type=jax.ShapeDtypeStruct((n, EMB), F32),
        scratch_types=[
            pltpu.VMEM((chunk,), jnp.int32),
            pltpu.VMEM((chunk, EMB), F32),
            pltpu.SemaphoreType.DMA,
        ],
    )
    def gather_k(idx_hbm, table_hbm, out_hbm, idx_v, rows_v, sem):
        wid = lax.axis_index("s") * nc + lax.axis_index("c")
        base = wid * b_per_w
        for c in range(n_chunks):
            off = base + c * chunk
            pltpu.sync_copy(idx_hbm.at[pl.ds(off, chunk)], idx_v)
            pltpu.async_copy(table_hbm.at[idx_v], rows_v, sem).wait()
            pltpu.sync_copy(rows_v, out_hbm.at[pl.ds(off, chunk)])

    return gather_k(idx, table)


def _gru_body(xc_ref, xn_ref, wih_ref, whh_ref, bih_ref, bhh_ref, out_ref,
              h_ref, gi_ref):
    t = pl.program_id(0)

    @pl.when(t == 0)
    def _():
        h_ref[...] = jnp.zeros_like(h_ref)
        gi_ref[...] = jnp.dot(xc_ref[0].astype(BF16), wih_ref[...],
                              preferred_element_type=F32)

    gi = gi_ref[...] + bih_ref[0]
    h = h_ref[...]
    gh = jnp.dot(h.astype(BF16), whh_ref[...],
                 preferred_element_type=F32) + bhh_ref[0]
    gi_ref[...] = jnp.dot(xn_ref[0].astype(BF16), wih_ref[...],
                          preferred_element_type=F32)
    r = jax.nn.sigmoid(gi[:, :HID] + gh[:, :HID])
    z = jax.nn.sigmoid(gi[:, HID:2 * HID] + gh[:, HID:2 * HID])
    n = jnp.tanh(gi[:, 2 * HID:] + r * gh[:, 2 * HID:])
    hn = n + z * (h - n)
    h_ref[...] = hn
    out_ref[...] = hn


def _gru(x, wih_t, whh_t, b_ih, b_hh):
    return pl.pallas_call(
        _gru_body,
        grid=(S,),
        in_specs=[
            pl.BlockSpec((1, B, EMB), lambda t: (t, 0, 0)),
            pl.BlockSpec((1, B, EMB),
                         lambda t: (jnp.minimum(t + 1, S - 1), 0, 0)),
            pl.BlockSpec((EMB, 3 * HID), lambda t: (0, 0)),
            pl.BlockSpec((HID, 3 * HID), lambda t: (0, 0)),
            pl.BlockSpec((1, 3 * HID), lambda t: (0, 0)),
            pl.BlockSpec((1, 3 * HID), lambda t: (0, 0)),
        ],
        out_specs=pl.BlockSpec((B, HID), lambda t: (0, 0)),
        out_shape=jax.ShapeDtypeStruct((B, HID), F32),
        scratch_shapes=[
            pltpu.VMEM((B, HID), F32),
            pltpu.VMEM((B, 3 * HID), F32),
        ],
        compiler_params=pltpu.CompilerParams(
            dimension_semantics=("arbitrary",),
        ),
    )(x, x, wih_t, whh_t, b_ih, b_hh)


_BB = 32


def _attn_body(img_ref, q_ref, w1a_ref, w1b_ref, b1_ref, w2_ref, ctx_ref):
    x = img_ref[...]
    recip = 1.0 / jnp.abs(x)
    nsum = jnp.sum(recip, axis=0)
    norm = jnp.maximum(1.0 / nsum, 1e-12)
    img = x / norm[None]

    q = q_ref[...]
    qw = jnp.dot(q.astype(BF16), w1b_ref[...],
                 preferred_element_type=F32)
    img2 = img.reshape(K * _BB, FEAT).astype(BF16)
    c1 = jnp.dot(img2, w1a_ref[...], preferred_element_type=F32)
    c1 = c1.reshape(K, _BB, HID) + qw[None] + b1_ref[0][None, None]
    c1 = jax.nn.relu(c1)

    att = jnp.sum(c1 * w2_ref[0][None, None, :], axis=2)
    att = att - jnp.max(att, axis=0)[None]
    e = jnp.exp(att)
    att = e / jnp.sum(e, axis=0)[None]

    ctx_ref[...] = jnp.sum(att[:, :, None] * img, axis=0)


def _attn(img_t, q_enc, w1a, w1b, b1, w2row):
    nblk = B // _BB
    const = lambda i: (0, 0)
    return pl.pallas_call(
        _attn_body,
        grid=(nblk,),
        in_specs=[
            pl.BlockSpec((K, _BB, FEAT), lambda i: (0, i, 0)),
            pl.BlockSpec((_BB, HID), lambda i: (i, 0)),
            pl.BlockSpec((FEAT, HID), const),
            pl.BlockSpec((HID, HID), const),
            pl.BlockSpec((1, HID), const),
            pl.BlockSpec((1, HID), const),
        ],
        out_specs=pl.BlockSpec((_BB, FEAT), lambda i: (i, 0)),
        out_shape=jax.ShapeDtypeStruct((B, FEAT), F32),
        compiler_params=pltpu.CompilerParams(
            dimension_semantics=("parallel",),
        ),
    )(img_t, q_enc, w1a, w1b, b1, w2row)


def _head_body(ctx_ref, q_ref, w3a_ref, w3b_ref, b3_ref, w4_ref, b4_ref,
               w5_ref, b5_ref, out_ref):
    h1 = jax.nn.relu(
        jnp.dot(ctx_ref[...].astype(BF16), w3a_ref[...],
                preferred_element_type=F32)
        + jnp.dot(q_ref[...].astype(BF16), w3b_ref[...],
                  preferred_element_type=F32)
        + b3_ref[0]
    )
    h2 = jax.nn.relu(jnp.dot(h1.astype(BF16), w4_ref[...],
                             preferred_element_type=F32) + b4_ref[0])
    out_ref[...] = (jnp.dot(h2.astype(BF16), w5_ref[...],
                            preferred_element_type=F32) + b5_ref[0])


def _head(ctx, q_enc, w3a, w3b, b3, w4, b4, w5, b5):
    return pl.pallas_call(
        _head_body,
        out_shape=jax.ShapeDtypeStruct((B, OUT), F32),
    )(ctx, q_enc, w3a, w3b, b3, w4, b4, w5, b5)


def kernel(question, image, wembed, gru_w_ih, gru_w_hh, gru_b_ih, gru_b_hh,
           W1, b1, W2, b2, W3, b3, W4, b4, W5, b5):
    idx = question.T.reshape(-1).astype(jnp.int32)
    img_t = jnp.transpose(image, (1, 0, 2))
    wih_t = gru_w_ih.T.astype(BF16)
    whh_t = gru_w_hh.T.astype(BF16)
    bih = gru_b_ih.reshape(1, 3 * HID)
    bhh = gru_b_hh.reshape(1, 3 * HID)
    w1a = W1[:FEAT].astype(BF16)
    w1b = W1[FEAT:].astype(BF16)
    w2row = W2.reshape(1, HID)
    w3a = W3[:FEAT].astype(BF16)
    w3b = W3[FEAT:].astype(BF16)
    b1r = b1.reshape(1, HID)
    b3r = b3.reshape(1, HID)
    b4r = b4.reshape(1, HID)
    b5r = b5.reshape(1, OUT)

    emb = _sc_gather(wembed, idx).reshape(S, B, EMB)

    q_enc = _gru(emb, wih_t, whh_t, bih, bhh)

    ctx = _attn(img_t, q_enc, w1a, w1b, b1r, w2row)
    return _head(ctx, q_enc, w3a, w3b, b3r, W4.astype(BF16), b4r,
                 W5.astype(BF16), b5r)
